# baseline (speedup 1.0000x reference)
import functools

import jax
import jax.numpy as jnp
from jax.experimental import pallas as pl
from jax.experimental.pallas import tpu as pltpu

_NUM_HEADS = 4
_SPARSE_RATIO = 0.0625
_NEG = -3.0e38


def _oddeven_merge_sort_pairs(n):
    net = []

    def merge(lo, m, r):
        step = r * 2
        if step < m:
            merge(lo, m, step)
            merge(lo + r, m, step)
            net.extend((i, i + r) for i in range(lo + r, lo + m - r, step))
        else:
            net.append((lo, lo + r))

    def sort(lo, m):
        if m > 1:
            h = m // 2
            sort(lo, h)
            sort(lo + h, h)
            merge(lo, m, 1)

    sort(0, n)
    return net


def _ctx_body(x_ref, o_ref):
    x = x_ref[...]
    avg = jnp.mean(x, axis=1)
    mx = jnp.max(x, axis=1)
    o_ref[...] = jnp.concatenate([avg, mx], axis=-1)


def _off_body(ctx_ref, w_ref, b_ref, o_ref):
    o_ref[...] = (
        jnp.dot(ctx_ref[...], w_ref[...], preferred_element_type=jnp.float32)
        + b_ref[...]
    )


def _main_body(x_ref, wpre_ref, bpre_ref, base_ref, off_ref, idx_ref, w_ref, *, k, inv_scale):
    x = x_ref[0]
    xp = jnp.dot(x, wpre_ref[...], preferred_element_type=jnp.float32) + bpre_ref[...]
    pro = base_ref[...] + off_ref[0]
    s = jax.lax.dot_general(
        xp, pro, (((1,), (1,)), ((), ())), preferred_element_type=jnp.float32
    ) * inv_scale
    nb, e = s.shape
    nlev = e // k
    lane = jax.lax.broadcasted_iota(jnp.int32, (nb, k), 1)
    _MINI = jnp.int32(-2147483648)
    _MAXI = jnp.int32(2147483647)

    si = jax.lax.bitcast_convert_type(s, jnp.int32)
    kall = si ^ ((si >> 31) & jnp.int32(0x7FFFFFFF))

    ks = [kall[:, i * k:(i + 1) * k] for i in range(nlev)]
    es = [lane + i * k for i in range(nlev)]
    for a, b in _oddeven_merge_sort_pairs(nlev):
        ka, kb, ea, eb = ks[a], ks[b], es[a], es[b]
        c = (ka > kb) | ((ka == kb) & (ea < eb))
        ks[a] = jnp.where(c, ka, kb)
        ks[b] = jnp.where(c, kb, ka)
        es[a] = jnp.where(c, ea, eb)
        es[b] = jnp.where(c, eb, ea)

    def _roll(v, sh):
        return pltpu.roll(v, sh, 1)

    def cond(carry):
        base = carry[0]
        return jnp.min(base) < k

    def body(carry):
        base, outk, oute = carry[0], carry[1], carry[2]
        ks = list(carry[3])
        es = list(carry[4])
        hs, hes, hss = ks[0], es[0], ks[1]
        for st in range(k.bit_length() - 1):
            for sub in range(st, -1, -1):
                d = 1 << sub
                up = (lane & d) == 0
                desc = (lane & (2 << st)) == 0
                updesc = up == desc
                hp = jnp.where(up, _roll(hs, k - d), _roll(hs, d))
                hep = jnp.where(up, _roll(hes, k - d), _roll(hes, d))
                hsp = jnp.where(up, _roll(hss, k - d), _roll(hss, d))
                gt = (hs > hp) | ((hs == hp) & (hes < hep))
                keep = gt == updesc
                hs = jnp.where(keep, hs, hp)
                hes = jnp.where(keep, hes, hep)
                hss = jnp.where(keep, hss, hsp)
        pf = jnp.where(lane == 0, _MINI, _roll(hss, 1))
        for t in range(k.bit_length() - 1):
            sh = 1 << t
            pf = jnp.maximum(pf, jnp.where(lane >= sh, _roll(pf, sh), _MINI))
        pop = (hs > pf) | (lane == 0)
        pop = pop & (lane < (k - base))
        p = jnp.sum(pop.astype(jnp.int32), axis=1, keepdims=True)
        lastk = jnp.min(jnp.where(pop, hs, _MAXI), axis=1, keepdims=True)
        laste = jnp.max(jnp.where(pop & (hs == lastk), hes, -1), axis=1, keepdims=True)
        rm = pop.astype(jnp.int32)
        rk, re = hs, hes
        for bit in [1 << t for t in range(k.bit_length() - 2, -1, -1)]:
            c = (base & bit) != 0
            rm = jnp.where(c, _roll(rm, bit), rm)
            rk = jnp.where(c, _roll(rk, bit), rk)
            re = jnp.where(c, _roll(re, bit), re)
        outk = jnp.where(rm != 0, rk, outk)
        oute = jnp.where(rm != 0, re, oute)
        pm = (ks[0] > lastk) | ((ks[0] == lastk) & (es[0] <= laste))
        for i in range(nlev - 1):
            ks[i] = jnp.where(pm, ks[i + 1], ks[i])
            es[i] = jnp.where(pm, es[i + 1], es[i])
        ks[nlev - 1] = jnp.where(pm, _MINI, ks[nlev - 1])
        return base + p, outk, oute, tuple(ks), tuple(es)

    base0 = jnp.zeros((nb, 1), jnp.int32)
    outk0 = jnp.zeros((nb, k), jnp.int32)
    oute0 = jnp.zeros((nb, k), jnp.int32)
    _, outk, ti, _, _ = jax.lax.while_loop(
        cond, body, (base0, outk0, oute0, tuple(ks), tuple(es)))
    tvi = outk ^ ((outk >> 31) & jnp.int32(0x7FFFFFFF))
    tv = jax.lax.bitcast_convert_type(tvi, jnp.float32)
    ex = jnp.exp(tv - tv[:, :1])
    w = ex / jnp.sum(ex, axis=1, keepdims=True)
    idx_ref[0] = ti
    w_ref[0] = w


def kernel(X, prototype_base, W_ctx, b_ctx, W_pre, b_pre):
    B, N, D = X.shape
    E = prototype_base.shape[0]
    k = max(1, int(E * _SPARSE_RATIO))
    inv_scale = 1.0 / (float(_NUM_HEADS) * float(D // _NUM_HEADS) ** 0.5)

    ctx = pl.pallas_call(
        _ctx_body,
        out_shape=jax.ShapeDtypeStruct((B, 2 * D), jnp.float32),
        in_specs=[pl.BlockSpec((B, N, D), lambda: (0, 0, 0))],
        out_specs=pl.BlockSpec((B, 2 * D), lambda: (0, 0)),
    )(X)

    ec = 16
    cw = (E * D) // ec
    off2 = pl.pallas_call(
        _off_body,
        grid=(ec,),
        out_shape=jax.ShapeDtypeStruct((B, E * D), jnp.float32),
        in_specs=[
            pl.BlockSpec((B, 2 * D), lambda i: (0, 0)),
            pl.BlockSpec((2 * D, cw), lambda i: (0, i)),
            pl.BlockSpec((1, cw), lambda i: (0, i)),
        ],
        out_specs=pl.BlockSpec((B, cw), lambda i: (0, i)),
    )(ctx, W_ctx, b_ctx.reshape(1, E * D))
    off3 = off2.reshape(B, E, D)

    nb = 256
    grid = (B, N // nb)
    idx, w = pl.pallas_call(
        functools.partial(_main_body, k=k, inv_scale=inv_scale),
        grid=grid,
        out_shape=(
            jax.ShapeDtypeStruct((B, N, k), jnp.int32),
            jax.ShapeDtypeStruct((B, N, k), jnp.float32),
        ),
        in_specs=[
            pl.BlockSpec((1, nb, D), lambda b, n: (b, n, 0)),
            pl.BlockSpec((D, D), lambda b, n: (0, 0)),
            pl.BlockSpec((1, D), lambda b, n: (0, 0)),
            pl.BlockSpec((E, D), lambda b, n: (0, 0)),
            pl.BlockSpec((1, E, D), lambda b, n: (b, 0, 0)),
        ],
        out_specs=(
            pl.BlockSpec((1, nb, k), lambda b, n: (b, n, 0)),
            pl.BlockSpec((1, nb, k), lambda b, n: (b, n, 0)),
        ),
    )(X, W_pre, b_pre.reshape(1, D), prototype_base, off3)
    return (idx, w, jnp.asarray(E, dtype=jnp.int32))


# speedup vs baseline: 1.0858x; 1.0858x over previous
import functools

import jax
import jax.numpy as jnp
from jax.experimental import pallas as pl
from jax.experimental.pallas import tpu as pltpu

_NUM_HEADS = 4
_SPARSE_RATIO = 0.0625
_NEG = -3.0e38


def _oddeven_merge_sort_pairs(n):
    net = []

    def merge(lo, m, r):
        step = r * 2
        if step < m:
            merge(lo, m, step)
            merge(lo + r, m, step)
            net.extend((i, i + r) for i in range(lo + r, lo + m - r, step))
        else:
            net.append((lo, lo + r))

    def sort(lo, m):
        if m > 1:
            h = m // 2
            sort(lo, h)
            sort(lo + h, h)
            merge(lo, m, 1)

    sort(0, n)
    return net


def _ctx_body(x_ref, o_ref):
    x = x_ref[...]
    avg = jnp.mean(x, axis=1)
    mx = jnp.max(x, axis=1)
    o_ref[...] = jnp.concatenate([avg, mx], axis=-1)


def _off_body(ctx_ref, w_ref, b_ref, o_ref):
    o_ref[...] = (
        jnp.dot(ctx_ref[...], w_ref[...], preferred_element_type=jnp.float32)
        + b_ref[...]
    )


def _main_body(x_ref, wpre_ref, bpre_ref, base_ref, off_ref, idx_ref, w_ref, *, k, inv_scale):
    x = x_ref[0]
    xp = jnp.dot(x, wpre_ref[...], preferred_element_type=jnp.float32) + bpre_ref[...]
    pro = base_ref[...] + off_ref[0]
    s = jax.lax.dot_general(
        xp, pro, (((1,), (1,)), ((), ())), preferred_element_type=jnp.float32
    ) * inv_scale
    nb, e = s.shape
    nlev = e // k
    lane = jax.lax.broadcasted_iota(jnp.int32, (nb, k), 1)
    _MINI = jnp.int32(-2147483648)
    _MAXI = jnp.int32(2147483647)

    si = jax.lax.bitcast_convert_type(s, jnp.int32)
    kall = si ^ ((si >> 31) & jnp.int32(0x7FFFFFFF))

    ks = [kall[:, i * k:(i + 1) * k] for i in range(nlev)]
    es = [lane + i * k for i in range(nlev)]
    for a, b in _oddeven_merge_sort_pairs(nlev):
        ka, kb, ea, eb = ks[a], ks[b], es[a], es[b]
        c = (ka > kb) | ((ka == kb) & (ea < eb))
        ks[a] = jnp.where(c, ka, kb)
        ks[b] = jnp.where(c, kb, ka)
        es[a] = jnp.where(c, ea, eb)
        es[b] = jnp.where(c, eb, ea)

    def _roll(v, sh):
        return pltpu.roll(v, sh, 1)

    def cond(carry):
        base = carry[0]
        return jnp.min(base) < k

    def body(carry):
        base, outk, oute = carry[0], carry[1], carry[2]
        ks = list(carry[3])
        es = list(carry[4])
        hs, hes = ks[0], es[0]
        sec = ks[1]
        m2 = jnp.max(sec, axis=1, keepdims=True)
        for st in range(k.bit_length() - 1):
            for sub in range(st, -1, -1):
                d = 1 << sub
                up = (lane & d) == 0
                desc = (lane & (2 << st)) == 0
                updesc = up == desc
                hp = jnp.where(up, _roll(hs, k - d), _roll(hs, d))
                hep = jnp.where(up, _roll(hes, k - d), _roll(hes, d))
                gt = (hs > hp) | ((hs == hp) & (hes < hep))
                keep = gt == updesc
                hs = jnp.where(keep, hs, hp)
                hes = jnp.where(keep, hes, hep)

        def pop_set(thr):
            pop = ((hs > thr) | (lane == 0)) & (lane < (k - base))
            lastk = jnp.min(jnp.where(pop, hs, _MAXI), axis=1, keepdims=True)
            laste = jnp.max(jnp.where(pop & (hs == lastk), hes, -1),
                            axis=1, keepdims=True)
            pm = (ks[0] > lastk) | ((ks[0] == lastk) & (es[0] <= laste))
            tmax = jnp.max(jnp.where(pm, sec, _MINI), axis=1, keepdims=True)
            return pop, lastk, laste, pm, tmax

        _, _, _, _, ta = pop_set(m2)
        _, _, _, _, tb = pop_set(ta)
        pop, lastk, laste, pm, _ = pop_set(jnp.maximum(tb, _MINI))
        p = jnp.sum(pop.astype(jnp.int32), axis=1, keepdims=True)
        rm = pop.astype(jnp.int32)
        rk, re = hs, hes
        for bit in [1 << t for t in range(k.bit_length() - 2, -1, -1)]:
            c = (base & bit) != 0
            rm = jnp.where(c, _roll(rm, bit), rm)
            rk = jnp.where(c, _roll(rk, bit), rk)
            re = jnp.where(c, _roll(re, bit), re)
        outk = jnp.where(rm != 0, rk, outk)
        oute = jnp.where(rm != 0, re, oute)
        for i in range(nlev - 1):
            ks[i] = jnp.where(pm, ks[i + 1], ks[i])
            es[i] = jnp.where(pm, es[i + 1], es[i])
        ks[nlev - 1] = jnp.where(pm, _MINI, ks[nlev - 1])
        return base + p, outk, oute, tuple(ks), tuple(es)

    base0 = jnp.zeros((nb, 1), jnp.int32)
    outk0 = jnp.zeros((nb, k), jnp.int32)
    oute0 = jnp.zeros((nb, k), jnp.int32)
    _, outk, ti, _, _ = jax.lax.while_loop(
        cond, body, (base0, outk0, oute0, tuple(ks), tuple(es)))
    tvi = outk ^ ((outk >> 31) & jnp.int32(0x7FFFFFFF))
    tv = jax.lax.bitcast_convert_type(tvi, jnp.float32)
    ex = jnp.exp(tv - tv[:, :1])
    w = ex / jnp.sum(ex, axis=1, keepdims=True)
    idx_ref[0] = ti
    w_ref[0] = w


def kernel(X, prototype_base, W_ctx, b_ctx, W_pre, b_pre):
    B, N, D = X.shape
    E = prototype_base.shape[0]
    k = max(1, int(E * _SPARSE_RATIO))
    inv_scale = 1.0 / (float(_NUM_HEADS) * float(D // _NUM_HEADS) ** 0.5)

    ctx = pl.pallas_call(
        _ctx_body,
        out_shape=jax.ShapeDtypeStruct((B, 2 * D), jnp.float32),
        in_specs=[pl.BlockSpec((B, N, D), lambda: (0, 0, 0))],
        out_specs=pl.BlockSpec((B, 2 * D), lambda: (0, 0)),
    )(X)

    ec = 16
    cw = (E * D) // ec
    off2 = pl.pallas_call(
        _off_body,
        grid=(ec,),
        out_shape=jax.ShapeDtypeStruct((B, E * D), jnp.float32),
        in_specs=[
            pl.BlockSpec((B, 2 * D), lambda i: (0, 0)),
            pl.BlockSpec((2 * D, cw), lambda i: (0, i)),
            pl.BlockSpec((1, cw), lambda i: (0, i)),
        ],
        out_specs=pl.BlockSpec((B, cw), lambda i: (0, i)),
    )(ctx, W_ctx, b_ctx.reshape(1, E * D))
    off3 = off2.reshape(B, E, D)

    nb = 256
    grid = (B, N // nb)
    idx, w = pl.pallas_call(
        functools.partial(_main_body, k=k, inv_scale=inv_scale),
        grid=grid,
        out_shape=(
            jax.ShapeDtypeStruct((B, N, k), jnp.int32),
            jax.ShapeDtypeStruct((B, N, k), jnp.float32),
        ),
        in_specs=[
            pl.BlockSpec((1, nb, D), lambda b, n: (b, n, 0)),
            pl.BlockSpec((D, D), lambda b, n: (0, 0)),
            pl.BlockSpec((1, D), lambda b, n: (0, 0)),
            pl.BlockSpec((E, D), lambda b, n: (0, 0)),
            pl.BlockSpec((1, E, D), lambda b, n: (b, 0, 0)),
        ],
        out_specs=(
            pl.BlockSpec((1, nb, k), lambda b, n: (b, n, 0)),
            pl.BlockSpec((1, nb, k), lambda b, n: (b, n, 0)),
        ),
    )(X, W_pre, b_pre.reshape(1, D), prototype_base, off3)
    return (idx, w, jnp.asarray(E, dtype=jnp.int32))


# speedup vs baseline: 1.5188x; 1.3988x over previous
import functools

import jax
import jax.numpy as jnp
from jax.experimental import pallas as pl
from jax.experimental.pallas import tpu as pltpu

_NUM_HEADS = 4
_SPARSE_RATIO = 0.0625
_NEG = -3.0e38


def _oddeven_merge_sort_pairs(n):
    net = []

    def merge(lo, m, r):
        step = r * 2
        if step < m:
            merge(lo, m, step)
            merge(lo + r, m, step)
            net.extend((i, i + r) for i in range(lo + r, lo + m - r, step))
        else:
            net.append((lo, lo + r))

    def sort(lo, m):
        if m > 1:
            h = m // 2
            sort(lo, h)
            sort(lo + h, h)
            merge(lo, m, 1)

    sort(0, n)
    return net


def _ctx_body(x_ref, o_ref):
    x = x_ref[...]
    avg = jnp.mean(x, axis=1)
    mx = jnp.max(x, axis=1)
    o_ref[...] = jnp.concatenate([avg, mx], axis=-1)


def _off_body(ctx_ref, w_ref, b_ref, o_ref):
    o_ref[...] = (
        jnp.dot(ctx_ref[...], w_ref[...], preferred_element_type=jnp.float32)
        + b_ref[...]
    )


def _main_body(x_ref, wpre_ref, bpre_ref, base_ref, off_ref, idx_ref, w_ref, *, k, inv_scale):
    x = x_ref[0]
    xp = jnp.dot(x, wpre_ref[...], preferred_element_type=jnp.float32) + bpre_ref[...]
    pro = base_ref[...] + off_ref[0]
    s = jax.lax.dot_general(
        xp, pro, (((1,), (1,)), ((), ())), preferred_element_type=jnp.float32
    ) * inv_scale
    nb, e = s.shape
    nlev = e // k
    lane = jax.lax.broadcasted_iota(jnp.int32, (nb, k), 1)
    _MINI = jnp.int32(-2147483648)
    _MAXI = jnp.int32(2147483647)

    si = jax.lax.bitcast_convert_type(s, jnp.int32)
    kall = si ^ ((si >> 31) & jnp.int32(0x7FFFFFFF))

    ks = [kall[:, i * k:(i + 1) * k] for i in range(nlev)]
    es = [lane + i * k for i in range(nlev)]
    for a, b in _oddeven_merge_sort_pairs(nlev):
        ka, kb, ea, eb = ks[a], ks[b], es[a], es[b]
        c = ka > kb
        ks[a] = jnp.where(c, ka, kb)
        ks[b] = jnp.where(c, kb, ka)
        es[a] = jnp.where(c, ea, eb)
        es[b] = jnp.where(c, eb, ea)

    def _roll(v, sh):
        return pltpu.roll(v, sh, 1)

    def cond(carry):
        base = carry[0]
        return jnp.min(base) < k

    def body(carry):
        base, outk, oute = carry[0], carry[1], carry[2]
        ks = list(carry[3])
        es = list(carry[4])
        hs, hes = ks[0], es[0]
        sec = ks[1]
        m2 = jnp.max(sec, axis=1, keepdims=True)
        for st in range(k.bit_length() - 1):
            for sub in range(st, -1, -1):
                d = 1 << sub
                up = (lane & d) == 0
                desc = (lane & (2 << st)) == 0
                updesc = up == desc
                hp = jnp.where(up, _roll(hs, k - d), _roll(hs, d))
                hep = jnp.where(up, _roll(hes, k - d), _roll(hes, d))
                gt = hs > hp
                keep = gt == updesc
                hs = jnp.where(keep, hs, hp)
                hes = jnp.where(keep, hes, hep)

        pop = ((hs > m2) | (lane == 0)) & (lane < (k - base))
        lastk = jnp.min(jnp.where(pop, hs, _MAXI), axis=1, keepdims=True)
        pm = ks[0] >= lastk
        p = jnp.sum(pop.astype(jnp.int32), axis=1, keepdims=True)
        rm = pop.astype(jnp.int32)
        rk, re = hs, hes
        for bit in [1 << t for t in range(k.bit_length() - 2, -1, -1)]:
            c = (base & bit) != 0
            rm = jnp.where(c, _roll(rm, bit), rm)
            rk = jnp.where(c, _roll(rk, bit), rk)
            re = jnp.where(c, _roll(re, bit), re)
        outk = jnp.where(rm != 0, rk, outk)
        oute = jnp.where(rm != 0, re, oute)
        for i in range(nlev - 1):
            ks[i] = jnp.where(pm, ks[i + 1], ks[i])
            es[i] = jnp.where(pm, es[i + 1], es[i])
        ks[nlev - 1] = jnp.where(pm, _MINI, ks[nlev - 1])
        return base + p, outk, oute, tuple(ks), tuple(es)

    base0 = jnp.zeros((nb, 1), jnp.int32)
    outk0 = jnp.zeros((nb, k), jnp.int32)
    oute0 = jnp.zeros((nb, k), jnp.int32)
    _, outk, ti, _, _ = jax.lax.while_loop(
        cond, body, (base0, outk0, oute0, tuple(ks), tuple(es)))
    tvi = outk ^ ((outk >> 31) & jnp.int32(0x7FFFFFFF))
    tv = jax.lax.bitcast_convert_type(tvi, jnp.float32)
    ex = jnp.exp(tv - tv[:, :1])
    w = ex / jnp.sum(ex, axis=1, keepdims=True)
    idx_ref[0] = ti
    w_ref[0] = w


def kernel(X, prototype_base, W_ctx, b_ctx, W_pre, b_pre):
    B, N, D = X.shape
    E = prototype_base.shape[0]
    k = max(1, int(E * _SPARSE_RATIO))
    inv_scale = 1.0 / (float(_NUM_HEADS) * float(D // _NUM_HEADS) ** 0.5)

    ctx = pl.pallas_call(
        _ctx_body,
        out_shape=jax.ShapeDtypeStruct((B, 2 * D), jnp.float32),
        in_specs=[pl.BlockSpec((B, N, D), lambda: (0, 0, 0))],
        out_specs=pl.BlockSpec((B, 2 * D), lambda: (0, 0)),
    )(X)

    ec = 16
    cw = (E * D) // ec
    off2 = pl.pallas_call(
        _off_body,
        grid=(ec,),
        out_shape=jax.ShapeDtypeStruct((B, E * D), jnp.float32),
        in_specs=[
            pl.BlockSpec((B, 2 * D), lambda i: (0, 0)),
            pl.BlockSpec((2 * D, cw), lambda i: (0, i)),
            pl.BlockSpec((1, cw), lambda i: (0, i)),
        ],
        out_specs=pl.BlockSpec((B, cw), lambda i: (0, i)),
    )(ctx, W_ctx, b_ctx.reshape(1, E * D))
    off3 = off2.reshape(B, E, D)

    nb = 256
    grid = (B, N // nb)
    idx, w = pl.pallas_call(
        functools.partial(_main_body, k=k, inv_scale=inv_scale),
        grid=grid,
        out_shape=(
            jax.ShapeDtypeStruct((B, N, k), jnp.int32),
            jax.ShapeDtypeStruct((B, N, k), jnp.float32),
        ),
        in_specs=[
            pl.BlockSpec((1, nb, D), lambda b, n: (b, n, 0)),
            pl.BlockSpec((D, D), lambda b, n: (0, 0)),
            pl.BlockSpec((1, D), lambda b, n: (0, 0)),
            pl.BlockSpec((E, D), lambda b, n: (0, 0)),
            pl.BlockSpec((1, E, D), lambda b, n: (b, 0, 0)),
        ],
        out_specs=(
            pl.BlockSpec((1, nb, k), lambda b, n: (b, n, 0)),
            pl.BlockSpec((1, nb, k), lambda b, n: (b, n, 0)),
        ),
    )(X, W_pre, b_pre.reshape(1, D), prototype_base, off3)
    return (idx, w, jnp.asarray(E, dtype=jnp.int32))
